# baseline (speedup 1.0000x reference)
import functools

import jax
import jax.numpy as jnp
from jax import lax
from jax.experimental import pallas as pl
from jax.experimental.pallas import tpu as pltpu
from jax.experimental.pallas import tpu_sc as plsc

B = 16384
D = 64
DP = 128
L = 16
NW = 32
ROWS_PER_W = B // NW
CHUNK = 128
N_CHUNKS = ROWS_PER_W // CHUNK
GROUPS = CHUNK // L


def _build_kernel():
    mesh = plsc.VectorSubcoreMesh(core_axis_name="c", subcore_axis_name="s")

    @functools.partial(
        pl.kernel,
        out_type=(
            jax.ShapeDtypeStruct((B,), jnp.float32),
            jax.ShapeDtypeStruct((B,), jnp.float32),
        ),
        mesh=mesh,
        compiler_params=pltpu.CompilerParams(
            needs_layout_passes=False, use_tc_tiling_on_sc=True),
        scratch_types=[
            pltpu.VMEM((N_CHUNKS, CHUNK), jnp.int32),
            pltpu.VMEM((N_CHUNKS, CHUNK), jnp.int32),
            pltpu.VMEM((N_CHUNKS, CHUNK), jnp.int32),
            pltpu.VMEM((CHUNK, D), jnp.float32),
            pltpu.VMEM((CHUNK, D), jnp.float32),
            pltpu.VMEM((CHUNK, DP), jnp.float32),
            pltpu.VMEM((CHUNK, D), jnp.float32),
            pltpu.VMEM((CHUNK, D), jnp.float32),
            pltpu.VMEM((CHUNK, DP), jnp.float32),
            pltpu.VMEM((ROWS_PER_W,), jnp.float32),
            pltpu.SemaphoreType.DMA,
            pltpu.SemaphoreType.DMA,
            pltpu.SemaphoreType.DMA,
            pltpu.SemaphoreType.DMA,
        ],
    )
    def distmult(pos_h, pos_t, pos_r, neg_h, neg_t, neg_r, ent, rel,
                 pos_out, neg_out,
                 idx_h, idx_t, idx_r,
                 h_a, t_a, r_a, h_b, t_b, r_b,
                 scores, sem_a, sem_b, rsem_a, rsem_b):
        wid = lax.axis_index("s") * 2 + lax.axis_index("c")
        base = wid * ROWS_PER_W
        row_ids = lax.iota(jnp.int32, L)
        rot = [(row_ids + (1 << p)) & (L - 1) for p in range(4)]

        def fetch(j, h_rows, t_rows, r_rows, sem, rsem):
            pltpu.async_copy(rel.at[idx_r.at[j]], r_rows, rsem)

            def fetch_body(g, _):
                hv = idx_h[j, pl.ds(g * L, L)]
                tv = idx_t[j, pl.ds(g * L, L)]
                for r in range(L):
                    row = g * L + r
                    eh = hv[r]
                    et = tv[r]
                    pltpu.async_copy(
                        ent.at[eh >> 4, eh & 15], h_rows.at[row], sem)
                    pltpu.async_copy(
                        ent.at[et >> 4, et & 15], t_rows.at[row], sem)
                return ()

            lax.fori_loop(0, GROUPS, fetch_body, ())

        def drain(h_rows, t_rows, r_rows, sem, rsem):
            def drain_body(rr, _):
                pltpu.make_async_copy(
                    ent.at[0, 0], h_rows.at[rr], sem).wait()
                pltpu.make_async_copy(
                    ent.at[0, 0], t_rows.at[rr], sem).wait()
                return ()

            lax.fori_loop(0, CHUNK, drain_body, ())
            pltpu.make_async_copy(
                rel.at[pl.ds(0, CHUNK)], r_rows, rsem).wait()

        def compute(j, h_rows, t_rows, r_rows):
            def group_body(g, _):
                tot = jnp.zeros((L,), jnp.float32)
                for r in range(L):
                    row = g * L + r
                    acc = (h_rows[row, pl.ds(0, L)]
                           * t_rows[row, pl.ds(0, L)]
                           * r_rows[row, pl.ds(0, L)])
                    for k in range(1, D // L):
                        acc += (h_rows[row, pl.ds(k * L, L)]
                                * t_rows[row, pl.ds(k * L, L)]
                                * r_rows[row, pl.ds(k * L, L)])
                    for p in range(4):
                        acc = acc + jnp.take(acc, rot[p])
                    tot = jnp.where(row_ids == r, acc, tot)
                scores[pl.ds(j * CHUNK + g * L, L)] = tot
                return ()

            lax.fori_loop(0, GROUPS, group_body, ())

        for h_hbm, t_hbm, r_hbm, out_hbm in (
            (pos_h, pos_t, pos_r, pos_out),
            (neg_h, neg_t, neg_r, neg_out),
        ):
            for j in range(N_CHUNKS):
                off = base + j * CHUNK
                pltpu.sync_copy(h_hbm.at[pl.ds(off, CHUNK)], idx_h.at[j])
                pltpu.sync_copy(t_hbm.at[pl.ds(off, CHUNK)], idx_t.at[j])
                pltpu.sync_copy(r_hbm.at[pl.ds(off, CHUNK)], idx_r.at[j])

            fetch(0, h_a, t_a, r_a, sem_a, rsem_a)

            def pair_body(i, _):
                j0 = 2 * i
                j1 = j0 + 1
                fetch(j1, h_b, t_b, r_b, sem_b, rsem_b)
                drain(h_a, t_a, r_a, sem_a, rsem_a)
                compute(j0, h_a, t_a, r_a)
                jn = jnp.minimum(j1 + 1, N_CHUNKS - 1)
                fetch(jn, h_a, t_a, r_a, sem_a, rsem_a)
                drain(h_b, t_b, r_b, sem_b, rsem_b)
                compute(j1, h_b, t_b, r_b)
                return ()

            lax.fori_loop(0, N_CHUNKS // 2, pair_body, ())
            drain(h_a, t_a, r_a, sem_a, rsem_a)
            pltpu.sync_copy(scores, out_hbm.at[pl.ds(base, ROWS_PER_W)])

    return distmult


_DISTMULT = _build_kernel()


def kernel(pos_h, pos_t, pos_r, neg_h, neg_t, neg_r, ent_emb, rel_emb):
    to_i32 = lambda x: jnp.asarray(x).astype(jnp.int32)
    rel_pad = jnp.pad(rel_emb, ((0, 0), (0, DP - D)))
    ent3 = ent_emb.reshape(ent_emb.shape[0] // 16, 16, ent_emb.shape[1])
    return _DISTMULT(
        to_i32(pos_h), to_i32(pos_t), to_i32(pos_r),
        to_i32(neg_h), to_i32(neg_t), to_i32(neg_r),
        ent3, rel_pad)


# speedup vs baseline: 1.0207x; 1.0207x over previous
import functools

import jax
import jax.numpy as jnp
from jax import lax
from jax.experimental import pallas as pl
from jax.experimental.pallas import tpu as pltpu
from jax.experimental.pallas import tpu_sc as plsc

B = 16384
D = 64
DP = 128
L = 16
NW = 32
ROWS_PER_W = B // NW
CHUNK = 128
N_CHUNKS = ROWS_PER_W // CHUNK
GROUPS = CHUNK // L


def _build_kernel():
    mesh = plsc.VectorSubcoreMesh(core_axis_name="c", subcore_axis_name="s")

    @functools.partial(
        pl.kernel,
        out_type=(
            jax.ShapeDtypeStruct((B,), jnp.float32),
            jax.ShapeDtypeStruct((B,), jnp.float32),
        ),
        mesh=mesh,
        compiler_params=pltpu.CompilerParams(
            needs_layout_passes=False, use_tc_tiling_on_sc=True),
        scratch_types=[
            pltpu.VMEM((N_CHUNKS, CHUNK), jnp.int32),
            pltpu.VMEM((N_CHUNKS, CHUNK), jnp.int32),
            pltpu.VMEM((N_CHUNKS, CHUNK), jnp.int32),
            pltpu.VMEM((CHUNK, D), jnp.float32),
            pltpu.VMEM((CHUNK, D), jnp.float32),
            pltpu.VMEM((CHUNK, DP), jnp.float32),
            pltpu.VMEM((ROWS_PER_W,), jnp.float32),
            pltpu.SemaphoreType.DMA,
            pltpu.SemaphoreType.DMA,
        ],
    )
    def distmult(pos_h, pos_t, pos_r, neg_h, neg_t, neg_r, ent, rel,
                 pos_out, neg_out,
                 idx_h, idx_t, idx_r, h_rows, t_rows, r_rows,
                 scores, sem, rsem):
        wid = lax.axis_index("s") * 2 + lax.axis_index("c")
        base = wid * ROWS_PER_W
        row_ids = lax.iota(jnp.int32, L)
        rot = [(row_ids + (1 << p)) & (L - 1) for p in range(4)]

        for h_hbm, t_hbm, r_hbm, out_hbm in (
            (pos_h, pos_t, pos_r, pos_out),
            (neg_h, neg_t, neg_r, neg_out),
        ):
            for j in range(N_CHUNKS):
                off = base + j * CHUNK
                pltpu.sync_copy(h_hbm.at[pl.ds(off, CHUNK)], idx_h.at[j])
                pltpu.sync_copy(t_hbm.at[pl.ds(off, CHUNK)], idx_t.at[j])
                pltpu.sync_copy(r_hbm.at[pl.ds(off, CHUNK)], idx_r.at[j])

            def chunk_body(j, _):
                rcopy = pltpu.async_copy(rel.at[idx_r.at[j]], r_rows, rsem)

                def fetch_body(g, _):
                    hv = idx_h[j, pl.ds(g * L, L)]
                    tv = idx_t[j, pl.ds(g * L, L)]
                    for r in range(L):
                        row = g * L + r
                        eh = hv[r]
                        et = tv[r]
                        pltpu.async_copy(
                            ent.at[eh >> 4, eh & 15], h_rows.at[row], sem)
                        pltpu.async_copy(
                            ent.at[et >> 4, et & 15], t_rows.at[row], sem)
                    return ()

                lax.fori_loop(0, GROUPS, fetch_body, ())

                def drain_body(rr, _):
                    pltpu.make_async_copy(
                        ent.at[0, 0], h_rows.at[rr], sem).wait()
                    pltpu.make_async_copy(
                        ent.at[0, 0], t_rows.at[rr], sem).wait()
                    return ()

                lax.fori_loop(0, CHUNK, drain_body, ())
                rcopy.wait()

                def group_body(g, _):
                    tot = jnp.zeros((L,), jnp.float32)
                    for r in range(L):
                        row = g * L + r
                        acc = (h_rows[row, pl.ds(0, L)]
                               * t_rows[row, pl.ds(0, L)]
                               * r_rows[row, pl.ds(0, L)])
                        for k in range(1, D // L):
                            acc += (h_rows[row, pl.ds(k * L, L)]
                                    * t_rows[row, pl.ds(k * L, L)]
                                    * r_rows[row, pl.ds(k * L, L)])
                        for p in range(4):
                            acc = acc + jnp.take(acc, rot[p])
                        tot = jnp.where(row_ids == r, acc, tot)
                    scores[pl.ds(j * CHUNK + g * L, L)] = tot
                    return ()

                lax.fori_loop(0, GROUPS, group_body, ())
                return ()

            lax.fori_loop(0, N_CHUNKS, chunk_body, ())
            pltpu.sync_copy(scores, out_hbm.at[pl.ds(base, ROWS_PER_W)])

    return distmult


_DISTMULT = _build_kernel()


def kernel(pos_h, pos_t, pos_r, neg_h, neg_t, neg_r, ent_emb, rel_emb):
    to_i32 = lambda x: jnp.asarray(x).astype(jnp.int32)
    rel_pad = jnp.pad(rel_emb, ((0, 0), (0, DP - D)))
    ent3 = ent_emb.reshape(ent_emb.shape[0] // 16, 16, ent_emb.shape[1])
    return _DISTMULT(
        to_i32(pos_h), to_i32(pos_t), to_i32(pos_r),
        to_i32(neg_h), to_i32(neg_t), to_i32(neg_r),
        ent3, rel_pad)
